# Initial kernel scaffold; baseline (speedup 1.0000x reference)
import jax
import jax.numpy as jnp
from jax.experimental import pallas as pl


def kernel(task_features, vm_features, task_edge_index, W1, att_src1, att_dst1, b1, W2, att_src2, att_dst2, b2):
    raise NotImplementedError("write your pallas kernel here")


# speedup vs baseline: 17.4311x; 17.4311x over previous
import functools

import jax
import jax.numpy as jnp
from jax import lax
from jax.experimental import pallas as pl
from jax.experimental.pallas import tpu as pltpu
from jax.experimental.pallas import tpu_sc as plsc

N_TASK = 2048
TASK_IN = 128
N_VM = 64
VM_IN = 32
EMB = 64
HEADS = 4
N_EDGE = 131072
E_TOT = N_EDGE + N_TASK

NC = 2
NS = 16
E_PAD = 135168
NSCAN = 33
SCHUNK = 2048
N_PAD = N_TASK + 8
ROWS_PER_TILE = N_TASK // NS
LISTCAP = 5120

LRELU = 0.2
EPS = 1e-16


def _tc_layer1(x_ref, w1_ref, attm_ref, h_ref, asd_ref, bnd_ref):
    h = jnp.dot(x_ref[...], w1_ref[...], preferred_element_type=jnp.float32)
    h_ref[...] = h
    asd = jnp.dot(h, attm_ref[...], preferred_element_type=jnp.float32)
    asd_ref[...] = asd
    m = jnp.max(asd[:, 0:HEADS], axis=0) + jnp.max(asd[:, HEADS:2 * HEADS], axis=0)
    m = jnp.maximum(m, LRELU * m)
    bnd_ref[...] = jnp.broadcast_to(m[:, None], (HEADS, 16))


def _tc_mid(acc_ref, b1_ref, w2_ref, attm2_ref, h2_ref, asd2_ref, bnd2_ref):
    a = acc_ref[0] + acc_ref[1]
    rows = a[:, 0:HEADS * EMB].reshape(N_TASK, HEADS, EMB)
    den = a[:, HEADS * EMB:HEADS * EMB + HEADS]
    t1 = rows / (den[..., None] + EPS)
    t1 = jnp.maximum(t1.reshape(N_TASK, HEADS * EMB) + b1_ref[...][None, :], 0.0)
    h2 = jnp.dot(t1, w2_ref[...], preferred_element_type=jnp.float32)
    h2_ref[...] = h2
    asd2 = jnp.dot(h2, attm2_ref[...], preferred_element_type=jnp.float32)
    asd2_ref[...] = asd2
    m = jnp.max(asd2[:, 0]) + jnp.max(asd2[:, 4])
    m = jnp.maximum(m, LRELU * m)
    bnd2_ref[...] = jnp.broadcast_to(m, (1, 16))


def _tc_final(acc_ref, b2_ref, t2_ref):
    a = acc_ref[0] + acc_ref[1]
    rows = a[:, 0:EMB]
    den = a[:, EMB:EMB + 1]
    t2 = rows / (den + EPS)
    t2_ref[...] = jnp.maximum(t2 + b2_ref[...][None, :], 0.0)


def _make_sc_edge_kernel(heads, cols):
    width = heads * cols
    accw = width + 16
    cgroups = cols // 16

    mesh = plsc.VectorSubcoreMesh(
        core_axis_name="c", subcore_axis_name="s", num_cores=NC, num_subcores=NS)

    @functools.partial(
        pl.kernel,
        out_type=jax.ShapeDtypeStruct((NC, N_TASK, accw), jnp.float32),
        mesh=mesh,
        compiler_params=pltpu.CompilerParams(
            needs_layout_passes=False, use_tc_tiling_on_sc=False),
        scratch_types=[
            pltpu.VMEM((ROWS_PER_TILE, accw), jnp.float32),
            pltpu.VMEM((N_PAD * 8,), jnp.float32),
            pltpu.VMEM((heads * 16,), jnp.float32),
            pltpu.VMEM((SCHUNK,), jnp.int32),
            pltpu.VMEM((SCHUNK,), jnp.int32),
            pltpu.VMEM((LISTCAP + 128,), jnp.int32),
            pltpu.VMEM((LISTCAP + 128,), jnp.int32),
            pltpu.VMEM((128,), jnp.int32),
            pltpu.VMEM((128, width), jnp.float32),
            pltpu.SemaphoreType.DMA,
        ],
    )
    def sc_edges(h_hbm, asd_hbm, bnd_hbm, src_hbm, dst_hbm, zero_hbm, out_hbm,
                 acc, asd_v, bnd_v, src_scan, dst_scan, src_l, dst_l,
                 idx_buf, g_buf, sem):
        c = lax.axis_index("c")
        s = lax.axis_index("s")
        base = s * ROWS_PER_TILE
        pltpu.sync_copy(zero_hbm.at[pl.ds(base, ROWS_PER_TILE)], acc)
        pltpu.sync_copy(asd_hbm, asd_v)
        pltpu.sync_copy(bnd_hbm, bnd_v)

        lanes = lax.iota(jnp.int32, 16)
        onehot = [(lanes == h).astype(jnp.float32) for h in range(heads)]

        def scan_body(sc, off):
            pltpu.sync_copy(src_hbm.at[c, sc], src_scan)
            pltpu.sync_copy(dst_hbm.at[c, sc], dst_scan)

            def scan_grp(k, off2):
                s16 = src_scan[pl.ds(k * 16, 16)]
                d16 = dst_scan[pl.ds(k * 16, 16)]
                m = (d16 >= base) & (d16 < base + ROWS_PER_TILE)
                plsc.store_compressed(src_l.at[pl.ds(off2, 16)], s16, mask=m)
                plsc.store_compressed(dst_l.at[pl.ds(off2, 16)], d16, mask=m)
                cnt = plsc.all_reduce_population_count(m)
                return off2 + cnt[0]

            return lax.fori_loop(0, SCHUNK // 16, scan_grp, off)

        off = lax.fori_loop(0, NSCAN, scan_body, jnp.int32(0))
        for k in range(8):
            src_l[pl.ds(off + k * 16, 16)] = jnp.full((16,), N_TASK, jnp.int32)
            dst_l[pl.ds(off + k * 16, 16)] = jnp.full((16,), base, jnp.int32)
        trips = (off + 127) // 128

        def trip_body(ci, carry):
            for k in range(8):
                idx_buf[pl.ds(k * 16, 16)] = src_l[pl.ds(ci * 128 + k * 16, 16)]
            pltpu.async_copy(h_hbm.at[idx_buf], g_buf, sem).wait()

            def grp_body(g, carry2):
                src16 = idx_buf[pl.ds(g * 16, 16)]
                dst16 = dst_l[pl.ds(ci * 128 + g * 16, 16)]
                ws = []
                for h in range(heads):
                    a_s = plsc.load_gather(asd_v, [src16 * 8 + h])
                    a_d = plsc.load_gather(asd_v, [dst16 * 8 + (4 + h)])
                    al = a_s + a_d
                    al = jnp.maximum(al, LRELU * al)
                    ws.append(jnp.exp(al - bnd_v[pl.ds(h * 16, 16)]))
                dloc = dst16 - base
                for e in range(16):
                    row = g * 16 + e
                    r = dloc[e]
                    wrow = jnp.zeros((16,), jnp.float32)
                    for h in range(heads):
                        wv = jnp.broadcast_to(ws[h][e], (16,))
                        wrow = wrow + wv * onehot[h]
                        for cg in range(cgroups):
                            col = h * cols + cg * 16
                            plsc.addupdate(
                                acc.at[r, pl.ds(col, 16)],
                                g_buf[row, pl.ds(col, 16)] * wv)
                    plsc.addupdate(acc.at[r, pl.ds(width, 16)], wrow)
                return carry2

            lax.fori_loop(0, 8, grp_body, 0)
            return carry

        lax.fori_loop(0, trips, trip_body, 0)
        pltpu.sync_copy(acc, out_hbm.at[c, pl.ds(base, ROWS_PER_TILE)])

    return sc_edges


_sc_edges_l1 = _make_sc_edge_kernel(HEADS, EMB)
_sc_edges_l2 = _make_sc_edge_kernel(1, EMB)


def kernel(task_features, vm_features, task_edge_index, W1, att_src1,
           att_dst1, b1, W2, att_src2, att_dst2, b2):
    idt = task_edge_index.dtype
    loop = jnp.arange(N_TASK, dtype=idt)
    npad = E_PAD - E_TOT
    src = jnp.concatenate(
        [task_edge_index[0], loop,
         jnp.full((npad,), N_TASK, idt)]).astype(jnp.int32)
    dst = jnp.concatenate(
        [task_edge_index[1], loop,
         (jnp.arange(npad) % N_TASK).astype(idt)]).astype(jnp.int32)
    src3d = src.reshape(NC, NSCAN, SCHUNK)
    dst3d = dst.reshape(NC, NSCAN, SCHUNK)

    a_s1 = att_src1.reshape(HEADS, EMB)
    a_d1 = att_dst1.reshape(HEADS, EMB)
    attm1 = jnp.zeros((HEADS * EMB, 8), jnp.float32)
    for h in range(HEADS):
        attm1 = attm1.at[h * EMB:(h + 1) * EMB, h].set(a_s1[h])
        attm1 = attm1.at[h * EMB:(h + 1) * EMB, 4 + h].set(a_d1[h])
    attm2 = jnp.zeros((EMB, 8), jnp.float32)
    attm2 = attm2.at[:, 0].set(att_src2.reshape(EMB))
    attm2 = attm2.at[:, 4].set(att_dst2.reshape(EMB))

    h1, asd1, bnd1 = pl.pallas_call(
        _tc_layer1,
        out_shape=[
            jax.ShapeDtypeStruct((N_TASK, HEADS * EMB), jnp.float32),
            jax.ShapeDtypeStruct((N_TASK, 8), jnp.float32),
            jax.ShapeDtypeStruct((HEADS, 16), jnp.float32),
        ],
    )(task_features, W1, attm1)

    h1p = jnp.pad(h1, ((0, N_PAD - N_TASK), (0, 0)))
    asd1p = jnp.pad(asd1, ((0, N_PAD - N_TASK), (0, 0)),
                    constant_values=-1e30).reshape(-1)
    zeros1 = jnp.zeros((N_TASK, HEADS * EMB + 16), jnp.float32)
    acc1 = _sc_edges_l1(h1p, asd1p, bnd1.reshape(-1), src3d, dst3d, zeros1)

    h2, asd2, bnd2 = pl.pallas_call(
        _tc_mid,
        out_shape=[
            jax.ShapeDtypeStruct((N_TASK, EMB), jnp.float32),
            jax.ShapeDtypeStruct((N_TASK, 8), jnp.float32),
            jax.ShapeDtypeStruct((1, 16), jnp.float32),
        ],
    )(acc1, b1, W2, attm2)

    h2p = jnp.pad(h2, ((0, N_PAD - N_TASK), (0, 0)))
    asd2p = jnp.pad(asd2, ((0, N_PAD - N_TASK), (0, 0)),
                    constant_values=-1e30).reshape(-1)
    zeros2 = jnp.zeros((N_TASK, EMB + 16), jnp.float32)
    acc2 = _sc_edges_l2(h2p, asd2p, bnd2.reshape(-1), src3d, dst3d, zeros2)

    t2 = pl.pallas_call(
        _tc_final,
        out_shape=jax.ShapeDtypeStruct((N_TASK, EMB), jnp.float32),
    )(acc2, b2)

    flat = t2.reshape(1, N_TASK * EMB)
    rep = jnp.broadcast_to(flat, (vm_features.shape[0], flat.shape[1]))
    return jnp.concatenate([rep, vm_features], axis=-1)
